# baseline (speedup 1.0000x reference)
import jax
import jax.numpy as jnp
from jax.experimental import pallas as pl
from jax.experimental.pallas import tpu as pltpu

_EPS = 1e-6
_NBUF = 3
_LOOKAHEAD = 2


def _ln_chunk(x, h, gamma, beta):
    s1 = jnp.sum(x, axis=-1, keepdims=True)
    s2 = jnp.sum(x * x, axis=-1, keepdims=True)
    mean = s1 * (1.0 / h)
    ssq = s2 - s1 * mean
    std = jnp.sqrt(ssq * (1.0 / max(h - 1, 1)))
    scale = gamma * pl.reciprocal(std + _EPS, approx=True)
    return (x - mean) * scale + beta


def _ln_manual_kernel(rows_per_core, chunk_rows, gamma_ref, beta_ref,
                      x_hbm, o_hbm, in_buf, out_buf, load_sem, store_sem):
    core = pl.program_id(0)
    base = core * rows_per_core
    nch = rows_per_core // chunk_rows
    h = in_buf.shape[-1]

    def load(i):
        slot = i % _NBUF
        pltpu.make_async_copy(
            x_hbm.at[pl.ds(base + i * chunk_rows, chunk_rows), :],
            in_buf.at[slot], load_sem.at[slot]).start()

    def store(i):
        slot = i % _NBUF
        pltpu.make_async_copy(
            out_buf.at[slot],
            o_hbm.at[pl.ds(base + i * chunk_rows, chunk_rows), :],
            store_sem.at[slot]).start()

    for i in range(min(_LOOKAHEAD, nch)):
        load(i)

    gamma = gamma_ref[0, 0]
    beta = beta_ref[0, 0]
    for i in range(nch):
        slot = i % _NBUF
        if i + _LOOKAHEAD < nch:
            load(i + _LOOKAHEAD)
        pltpu.make_async_copy(
            x_hbm.at[pl.ds(base + i * chunk_rows, chunk_rows), :],
            in_buf.at[slot], load_sem.at[slot]).wait()
        if i >= _NBUF:
            pltpu.make_async_copy(
                out_buf.at[slot],
                o_hbm.at[pl.ds(base + (i - _NBUF) * chunk_rows, chunk_rows), :],
                store_sem.at[slot]).wait()
        out_buf[slot] = _ln_chunk(in_buf[slot], h, gamma, beta)
        store(i)

    for i in range(max(0, nch - _NBUF), nch):
        slot = i % _NBUF
        pltpu.make_async_copy(
            out_buf.at[slot],
            o_hbm.at[pl.ds(base + i * chunk_rows, chunk_rows), :],
            store_sem.at[slot]).wait()


def _layer_norm(x, gamma, beta, *, chunk_rows=2048):
    orig_shape = x.shape
    H = orig_shape[-1]
    xf = x.reshape(-1, H)
    R = xf.shape[0]
    dtype = x.dtype

    g = jnp.asarray(gamma, jnp.float32).reshape(1, 1)
    b = jnp.asarray(beta, jnp.float32).reshape(1, 1)

    rows_per_core = -(-R // 2)
    chunk_rows = min(chunk_rows, max(8, -(-rows_per_core // 8) * 8))
    nch = -(-rows_per_core // chunk_rows)
    rows_per_core = nch * chunk_rows
    padded_rows = 2 * rows_per_core
    if padded_rows != R:
        xf = jnp.pad(xf, ((0, padded_rows - R), (0, 0)))

    import functools
    body = functools.partial(_ln_manual_kernel, rows_per_core, chunk_rows)
    smem = pl.BlockSpec(memory_space=pltpu.MemorySpace.SMEM)
    hbm = pl.BlockSpec(memory_space=pl.ANY)
    out = pl.pallas_call(
        body,
        out_shape=jax.ShapeDtypeStruct((padded_rows, H), dtype),
        grid=(2,),
        in_specs=[smem, smem, hbm],
        out_specs=hbm,
        scratch_shapes=[
            pltpu.VMEM((_NBUF, chunk_rows, H), jnp.float32),
            pltpu.VMEM((_NBUF, chunk_rows, H), jnp.float32),
            pltpu.SemaphoreType.DMA((_NBUF,)),
            pltpu.SemaphoreType.DMA((_NBUF,)),
        ],
        compiler_params=pltpu.CompilerParams(
            dimension_semantics=("parallel",),
            vmem_limit_bytes=64 << 20,
        ),
    )(g, b, xf)

    return out[:R].reshape(orig_shape)


def kernel(x, gamma, beta):
    return _layer_norm(x, gamma, beta)


# speedup vs baseline: 1.1040x; 1.1040x over previous
import jax
import jax.numpy as jnp
from jax.experimental import pallas as pl
from jax.experimental.pallas import tpu as pltpu

_EPS = 1e-6
_NBUF = 8
_LOOKAHEAD = 6


def _ln_chunk(x, h, gamma, beta):
    s1 = jnp.sum(x, axis=-1, keepdims=True)
    s2 = jnp.sum(x * x, axis=-1, keepdims=True)
    mean = s1 * (1.0 / h)
    ssq = s2 - s1 * mean
    std = jnp.sqrt(ssq * (1.0 / max(h - 1, 1)))
    scale = gamma * pl.reciprocal(std + _EPS, approx=True)
    return (x - mean) * scale + beta


def _ln_manual_kernel(rows_per_core, chunk_rows, gamma_ref, beta_ref,
                      x_hbm, o_hbm, in_buf, out_buf, load_sem, store_sem):
    core = pl.program_id(0)
    base = core * rows_per_core
    nch = rows_per_core // chunk_rows
    h = in_buf.shape[-1]

    def load(i):
        slot = i % _NBUF
        pltpu.make_async_copy(
            x_hbm.at[pl.ds(base + i * chunk_rows, chunk_rows), :],
            in_buf.at[slot], load_sem.at[slot]).start()

    def store(i):
        slot = i % _NBUF
        pltpu.make_async_copy(
            out_buf.at[slot],
            o_hbm.at[pl.ds(base + i * chunk_rows, chunk_rows), :],
            store_sem.at[slot]).start()

    for i in range(min(_LOOKAHEAD, nch)):
        load(i)

    gamma = gamma_ref[0, 0]
    beta = beta_ref[0, 0]
    for i in range(nch):
        slot = i % _NBUF
        if i + _LOOKAHEAD < nch:
            load(i + _LOOKAHEAD)
        pltpu.make_async_copy(
            x_hbm.at[pl.ds(base + i * chunk_rows, chunk_rows), :],
            in_buf.at[slot], load_sem.at[slot]).wait()
        if i >= _NBUF:
            pltpu.make_async_copy(
                out_buf.at[slot],
                o_hbm.at[pl.ds(base + (i - _NBUF) * chunk_rows, chunk_rows), :],
                store_sem.at[slot]).wait()
        out_buf[slot] = _ln_chunk(in_buf[slot], h, gamma, beta)
        store(i)

    for i in range(max(0, nch - _NBUF), nch):
        slot = i % _NBUF
        pltpu.make_async_copy(
            out_buf.at[slot],
            o_hbm.at[pl.ds(base + i * chunk_rows, chunk_rows), :],
            store_sem.at[slot]).wait()


def _layer_norm(x, gamma, beta, *, chunk_rows=512):
    orig_shape = x.shape
    H = orig_shape[-1]
    xf = x.reshape(-1, H)
    R = xf.shape[0]
    dtype = x.dtype

    g = jnp.asarray(gamma, jnp.float32).reshape(1, 1)
    b = jnp.asarray(beta, jnp.float32).reshape(1, 1)

    rows_per_core = -(-R // 2)
    chunk_rows = min(chunk_rows, max(8, -(-rows_per_core // 8) * 8))
    nch = -(-rows_per_core // chunk_rows)
    rows_per_core = nch * chunk_rows
    padded_rows = 2 * rows_per_core
    if padded_rows != R:
        xf = jnp.pad(xf, ((0, padded_rows - R), (0, 0)))

    import functools
    body = functools.partial(_ln_manual_kernel, rows_per_core, chunk_rows)
    smem = pl.BlockSpec(memory_space=pltpu.MemorySpace.SMEM)
    hbm = pl.BlockSpec(memory_space=pl.ANY)
    out = pl.pallas_call(
        body,
        out_shape=jax.ShapeDtypeStruct((padded_rows, H), dtype),
        grid=(2,),
        in_specs=[smem, smem, hbm],
        out_specs=hbm,
        scratch_shapes=[
            pltpu.VMEM((_NBUF, chunk_rows, H), jnp.float32),
            pltpu.VMEM((_NBUF, chunk_rows, H), jnp.float32),
            pltpu.SemaphoreType.DMA((_NBUF,)),
            pltpu.SemaphoreType.DMA((_NBUF,)),
        ],
        compiler_params=pltpu.CompilerParams(
            dimension_semantics=("parallel",),
            vmem_limit_bytes=64 << 20,
        ),
    )(g, b, xf)

    return out[:R].reshape(orig_shape)


def kernel(x, gamma, beta):
    return _layer_norm(x, gamma, beta)


# speedup vs baseline: 1.1531x; 1.0445x over previous
import functools

import jax
import jax.numpy as jnp
from jax.experimental import pallas as pl
from jax.experimental.pallas import tpu as pltpu

_EPS = 1e-6
_NBUF = 6
_LOOKAHEAD = 5
_BODY_ROWS = 1024


def _schedule(rows):
    sched = []
    rem = rows
    while rem > 1536:
        sched.append(_BODY_ROWS)
        rem -= _BODY_ROWS
    size = 512
    while rem > 0:
        c = min(size, rem)
        sched.append(c)
        rem -= c
        if size > 256:
            size //= 2
    offs = []
    o = 0
    for c in sched:
        offs.append(o)
        o += c
    return tuple(zip(offs, sched))


def _ln_chunk(x, h, gamma, beta):
    s1 = jnp.sum(x, axis=-1, keepdims=True)
    s2 = jnp.sum(x * x, axis=-1, keepdims=True)
    mean = s1 * (1.0 / h)
    ssq = s2 - s1 * mean
    std = jnp.sqrt(ssq * (1.0 / max(h - 1, 1)))
    scale = gamma * pl.reciprocal(std + _EPS, approx=True)
    return (x - mean) * scale + beta


def _ln_manual_kernel(rows_per_core, sched, gamma_ref, beta_ref,
                      x_hbm, o_hbm, in_buf, out_buf, load_sem, store_sem):
    core = pl.program_id(0)
    base = core * rows_per_core
    nch = len(sched)
    h = in_buf.shape[-1]

    def load_copy(i):
        off, c = sched[i]
        slot = i % _NBUF
        return pltpu.make_async_copy(
            x_hbm.at[pl.ds(base + off, c), :],
            in_buf.at[slot, pl.ds(0, c), :], load_sem.at[slot])

    def store_copy(i):
        off, c = sched[i]
        slot = i % _NBUF
        return pltpu.make_async_copy(
            out_buf.at[slot, pl.ds(0, c), :],
            o_hbm.at[pl.ds(base + off, c), :], store_sem.at[slot])

    for i in range(min(_LOOKAHEAD, nch)):
        load_copy(i).start()

    gamma = gamma_ref[0, 0]
    beta = beta_ref[0, 0]
    for i in range(nch):
        _, c = sched[i]
        slot = i % _NBUF
        if i + _LOOKAHEAD < nch:
            load_copy(i + _LOOKAHEAD).start()
        load_copy(i).wait()
        if i >= _NBUF:
            store_copy(i - _NBUF).wait()
        out_buf[slot, :c] = _ln_chunk(in_buf[slot, :c], h, gamma, beta)
        store_copy(i).start()

    for i in range(max(0, nch - _NBUF), nch):
        store_copy(i).wait()


def _layer_norm(x, gamma, beta):
    orig_shape = x.shape
    H = orig_shape[-1]
    xf = x.reshape(-1, H)
    R = xf.shape[0]
    dtype = x.dtype

    g = jnp.asarray(gamma, jnp.float32).reshape(1, 1)
    b = jnp.asarray(beta, jnp.float32).reshape(1, 1)

    rows_per_core = -(-(-(-R // 2)) // 8) * 8
    padded_rows = 2 * rows_per_core
    if padded_rows != R:
        xf = jnp.pad(xf, ((0, padded_rows - R), (0, 0)))
    sched = _schedule(rows_per_core)
    max_chunk = max(c for _, c in sched)

    body = functools.partial(_ln_manual_kernel, rows_per_core, sched)
    smem = pl.BlockSpec(memory_space=pltpu.MemorySpace.SMEM)
    hbm = pl.BlockSpec(memory_space=pl.ANY)
    out = pl.pallas_call(
        body,
        out_shape=jax.ShapeDtypeStruct((padded_rows, H), dtype),
        grid=(2,),
        in_specs=[smem, smem, hbm],
        out_specs=hbm,
        scratch_shapes=[
            pltpu.VMEM((_NBUF, max_chunk, H), jnp.float32),
            pltpu.VMEM((_NBUF, max_chunk, H), jnp.float32),
            pltpu.SemaphoreType.DMA((_NBUF,)),
            pltpu.SemaphoreType.DMA((_NBUF,)),
        ],
        compiler_params=pltpu.CompilerParams(
            dimension_semantics=("parallel",),
            vmem_limit_bytes=64 << 20,
        ),
    )(g, b, xf)

    return out[:R].reshape(orig_shape)


def kernel(x, gamma, beta):
    return _layer_norm(x, gamma, beta)
